# Initial kernel scaffold; baseline (speedup 1.0000x reference)
import jax
import jax.numpy as jnp
from jax.experimental import pallas as pl


def kernel(B0, Bi, W, alpha, h):
    raise NotImplementedError("write your pallas kernel here")


# speedup vs baseline: 4.5810x; 4.5810x over previous
import jax
import jax.numpy as jnp
from jax.experimental import pallas as pl
from jax.experimental.pallas import tpu as pltpu

F = 40
N = 40
E = 64
BB = 16


def _fusion_kernel(alpha_t_ref, bi_t_ref, b0_t_ref, w_ref, h_ref, out_ref):

    a = jax.lax.dot_general(
        alpha_t_ref[...], bi_t_ref[...],
        dimension_numbers=(((2,), (0,)), ((), ())),
        preferred_element_type=jnp.float32,
    )

    wh = w_ref[...] * h_ref[...][:, None, :]
    g = jax.lax.dot_general(
        a, wh,
        dimension_numbers=(((3,), (2,)), ((0,), (0,))),
        preferred_element_type=jnp.float32,
    )

    acc = jnp.sum(g * b0_t_ref[...][None, :, :, :], axis=1)
    out_ref[...] = jnp.transpose(acc, (1, 0, 2))


def kernel(B0, Bi, W, alpha, h):
    batch = B0.shape[0]
    alpha_t = jnp.transpose(alpha, (2, 0, 1))
    bi_t = jnp.transpose(Bi, (1, 0, 2))
    b0_t = jnp.transpose(B0, (1, 0, 2))
    h2 = h[..., 0]

    grid = (batch // BB,)
    out = pl.pallas_call(
        _fusion_kernel,
        grid=grid,
        in_specs=[
            pl.BlockSpec((N, F, F), lambda j: (0, 0, 0)),
            pl.BlockSpec((F, BB, E), lambda j: (0, j, 0)),
            pl.BlockSpec((F, BB, E), lambda j: (0, j, 0)),
            pl.BlockSpec((N, E, E), lambda j: (0, 0, 0)),
            pl.BlockSpec((N, E), lambda j: (0, 0)),
        ],
        out_specs=pl.BlockSpec((BB, N, E), lambda j: (j, 0, 0)),
        out_shape=jax.ShapeDtypeStruct((batch, N, E), jnp.float32),
        compiler_params=pltpu.CompilerParams(
            dimension_semantics=("parallel",),
        ),
    )(alpha_t, bi_t, b0_t, W, h2)
    return out
